# Initial kernel scaffold; baseline (speedup 1.0000x reference)
import jax
import jax.numpy as jnp
from jax.experimental import pallas as pl


def kernel(location, embedding):
    raise NotImplementedError("write your pallas kernel here")


# speedup vs baseline: 2.9807x; 2.9807x over previous
import functools

import jax
import jax.numpy as jnp
from jax import lax
from jax.experimental import pallas as pl
from jax.experimental.pallas import tpu as pltpu
from jax.experimental.pallas import tpu_sc as plsc

D = 32
NC = 2
NS = 16
NW = NC * NS
CHUNK = 128
K = 8
GROUP = CHUNK * K


def _make_gather(B: int):
    assert B % (NW * GROUP) == 0
    per_w = B // NW
    n_chunks = per_w // CHUNK
    n_groups = per_w // GROUP

    mesh = plsc.VectorSubcoreMesh(core_axis_name="c", subcore_axis_name="s")

    @functools.partial(
        pl.kernel,
        mesh=mesh,
        compiler_params=pltpu.CompilerParams(use_tc_tiling_on_sc=False),
        out_type=jax.ShapeDtypeStruct((B, D), jnp.float32),
        scratch_types=[
            pltpu.VMEM((n_chunks, CHUNK), jnp.int32),
            pltpu.VMEM((GROUP, D), jnp.float32),
            pltpu.SemaphoreType.DMA,
        ],
    )
    def gather(idx_hbm, table_hbm, out_hbm, idx_v, rows_v, sem):
        wid = lax.axis_index("s") * NC + lax.axis_index("c")
        base = wid * per_w
        pltpu.sync_copy(idx_hbm.at[wid], idx_v)

        @pl.loop(0, n_groups)
        def _(g):
            handles = []
            for k in range(K):
                h = pltpu.async_copy(
                    table_hbm.at[idx_v.at[g * K + k]],
                    rows_v.at[pl.ds(k * CHUNK, CHUNK)],
                    sem,
                )
                handles.append(h)
            for h in handles:
                h.wait()
            pltpu.sync_copy(rows_v, out_hbm.at[pl.ds(base + g * GROUP, GROUP)])

    return gather


@jax.jit
def _run(location, embedding):
    rows, cols = location.shape
    B = rows * cols
    idx = location.reshape(NW, B // (NW * CHUNK), CHUNK).astype(jnp.int32)
    out = _make_gather(B)(idx, embedding)
    return out.reshape(rows, cols, D)


def kernel(location, embedding):
    return _run(location, embedding)


# speedup vs baseline: 3.0023x; 1.0072x over previous
import functools

import jax
import jax.numpy as jnp
from jax import lax
from jax.experimental import pallas as pl
from jax.experimental.pallas import tpu as pltpu
from jax.experimental.pallas import tpu_sc as plsc

D = 32
NC = 2
NS = 16
NW = NC * NS
CHUNK = 128
K = 10
GROUP = CHUNK * K


def _make_gather(B: int):
    assert B % (NW * GROUP) == 0
    per_w = B // NW
    n_chunks = per_w // CHUNK
    n_groups = per_w // GROUP
    assert n_groups >= 4 and (n_groups - 2) % 2 == 0

    mesh = plsc.VectorSubcoreMesh(core_axis_name="c", subcore_axis_name="s")

    @functools.partial(
        pl.kernel,
        mesh=mesh,
        compiler_params=pltpu.CompilerParams(use_tc_tiling_on_sc=False),
        out_type=jax.ShapeDtypeStruct((B, D), jnp.float32),
        scratch_types=[
            pltpu.VMEM((n_chunks, CHUNK), jnp.int32),
            pltpu.VMEM((GROUP, D), jnp.float32),
            pltpu.VMEM((GROUP, D), jnp.float32),
            pltpu.SemaphoreType.DMA,
            pltpu.SemaphoreType.DMA,
            pltpu.SemaphoreType.DMA,
            pltpu.SemaphoreType.DMA,
        ],
    )
    def gather(idx_hbm, table_hbm, out_hbm, idx_v, rows0, rows1,
               gsem0, gsem1, wsem0, wsem1):
        wid = lax.axis_index("s") * NC + lax.axis_index("c")
        base = wid * per_w
        rows = (rows0, rows1)
        gsem = (gsem0, gsem1)
        wsem = (wsem0, wsem1)

        pltpu.sync_copy(idx_hbm.at[wid], idx_v)

        def issue_gathers(g, a):
            for k in range(K):
                pltpu.async_copy(
                    table_hbm.at[idx_v.at[g * K + k]],
                    rows[a].at[pl.ds(k * CHUNK, CHUNK)],
                    gsem[a],
                )

        def wait_gathers(a):
            pltpu.make_async_copy(
                out_hbm.at[pl.ds(0, GROUP)], rows[a], gsem[a]).wait()

        def issue_write(g, a):
            pltpu.async_copy(
                rows[a], out_hbm.at[pl.ds(base + g * GROUP, GROUP)], wsem[a])

        def wait_write(a):
            pltpu.make_async_copy(
                rows[a], out_hbm.at[pl.ds(0, GROUP)], wsem[a]).wait()

        issue_gathers(0, 0)
        wait_gathers(0)
        issue_write(0, 0)
        issue_gathers(1, 1)

        @pl.loop(1, n_groups - 1, step=2)
        def _(g0):
            for a in (1, 0):
                g = g0 if a == 1 else g0 + 1
                wait_gathers(a)
                issue_write(g, a)
                wait_write(1 - a)
                issue_gathers(g + 1, 1 - a)

        wait_gathers(1)
        issue_write(n_groups - 1, 1)
        wait_write(0)
        wait_write(1)

    return gather


@jax.jit
def _run(location, embedding):
    rows, cols = location.shape
    B = rows * cols
    idx = location.reshape(NW, B // (NW * CHUNK), CHUNK).astype(jnp.int32)
    out = _make_gather(B)(idx, embedding)
    return out.reshape(rows, cols, D)


def kernel(location, embedding):
    return _run(location, embedding)


# speedup vs baseline: 6.1224x; 2.0393x over previous
import functools

import jax
import jax.numpy as jnp
from jax import lax
from jax.experimental import pallas as pl
from jax.experimental.pallas import tpu as pltpu
from jax.experimental.pallas import tpu_sc as plsc

D = 32
NC = 2
NS = 16
NW = NC * NS
RK = 16


def _make_gather(n_rows: int, n_cols: int):
    assert n_rows % (NW * RK) == 0
    rows_per_w = n_rows // NW
    n_groups = rows_per_w // RK
    assert n_groups >= 4 and (n_groups - 2) % 2 == 0

    mesh = plsc.VectorSubcoreMesh(core_axis_name="c", subcore_axis_name="s")

    @functools.partial(
        pl.kernel,
        mesh=mesh,
        compiler_params=pltpu.CompilerParams(use_tc_tiling_on_sc=False),
        out_type=jax.ShapeDtypeStruct((n_rows, n_cols, D), jnp.float32),
        scratch_types=[
            pltpu.VMEM((rows_per_w, n_cols), jnp.int32),
            pltpu.VMEM((RK, n_cols, D), jnp.float32),
            pltpu.VMEM((RK, n_cols, D), jnp.float32),
            pltpu.SemaphoreType.DMA,
            pltpu.SemaphoreType.DMA,
            pltpu.SemaphoreType.DMA,
            pltpu.SemaphoreType.DMA,
        ],
    )
    def gather(loc_hbm, table_hbm, out_hbm, idx_v, rows0, rows1,
               gsem0, gsem1, wsem0, wsem1):
        wid = lax.axis_index("s") * NC + lax.axis_index("c")
        base = wid * rows_per_w
        rows = (rows0, rows1)
        gsem = (gsem0, gsem1)
        wsem = (wsem0, wsem1)

        pltpu.sync_copy(loc_hbm.at[pl.ds(base, rows_per_w)], idx_v)

        def issue_gathers(g, a):
            for k in range(RK):
                pltpu.async_copy(
                    table_hbm.at[idx_v.at[g * RK + k]],
                    rows[a].at[k],
                    gsem[a],
                )

        def wait_gathers(a):
            pltpu.make_async_copy(
                out_hbm.at[pl.ds(0, RK)], rows[a], gsem[a]).wait()

        def issue_write(g, a):
            pltpu.async_copy(
                rows[a], out_hbm.at[pl.ds(base + g * RK, RK)], wsem[a])

        def wait_write(a):
            pltpu.make_async_copy(
                rows[a], out_hbm.at[pl.ds(0, RK)], wsem[a]).wait()

        issue_gathers(0, 0)
        wait_gathers(0)
        issue_write(0, 0)
        issue_gathers(1, 1)

        @pl.loop(1, n_groups - 1, step=2)
        def _(g0):
            for a in (1, 0):
                g = g0 if a == 1 else g0 + 1
                wait_gathers(a)
                issue_write(g, a)
                wait_write(1 - a)
                issue_gathers(g + 1, 1 - a)

        wait_gathers(1)
        issue_write(n_groups - 1, 1)
        wait_write(0)
        wait_write(1)

    return gather


@jax.jit
def _run(location, embedding):
    n_rows, n_cols = location.shape
    return _make_gather(n_rows, n_cols)(location.astype(jnp.int32), embedding)


def kernel(location, embedding):
    return _run(location, embedding)
